# baseline (speedup 1.0000x reference)
import functools

import jax
import jax.numpy as jnp
import numpy as np
from jax import lax
from jax.experimental import pallas as pl
from jax.experimental.pallas import tpu as pltpu
from jax.experimental.pallas import tpu_sc as plsc

B = 16384
TEMPERATURE = 0.05
MAX_SCALE = 1.0 - 0.001

NC = 2
NS = 16
NW = NC * NS
TPW = B // NW
IPW = 3 * TPW
IDX_ROWS = IPW // 128
CHUNKS = TPW // 16


def _sc_gather(ids_t_flat, table_flat):
  mesh = plsc.VectorSubcoreMesh(
      core_axis_name="c", subcore_axis_name="s", num_cores=NC, num_subcores=NS)

  @functools.partial(
      pl.kernel,
      out_type=jax.ShapeDtypeStruct((6 * B,), jnp.float32),
      mesh=mesh,
      scratch_types=[
          pltpu.VMEM((IPW,), jnp.int32),
          pltpu.VMEM((2 * IPW,), jnp.int32),
          pltpu.VMEM((6 * TPW,), jnp.float32),
          pltpu.SemaphoreType.DMA,
      ],
  )
  def gather_kernel(ids_hbm, table_hbm, out_hbm, ids_v, cidx_v, comp_v, sem):
    wid = lax.axis_index("s") * NC + lax.axis_index("c")
    base = wid * TPW
    for j in range(3):
      pltpu.sync_copy(ids_hbm.at[pl.ds(j * B + base, TPW)],
                      ids_v.at[pl.ds(j * TPW, TPW)])

    def body(p, carry):
      for j in range(3):
        idv = ids_v[pl.ds(j * TPW + p * 16, 16)]
        two = idv * 2
        cidx_v[pl.ds((2 * j) * TPW + p * 16, 16)] = two
        cidx_v[pl.ds((2 * j + 1) * TPW + p * 16, 16)] = two + 1
      return carry

    lax.fori_loop(0, TPW // 16, body, 0, unroll=False)

    copies = [
        pltpu.async_copy(
            table_hbm.at[cidx_v.at[pl.ds(q * 128, 128)]],
            comp_v.at[pl.ds(q * 128, 128)], sem)
        for q in range(6 * TPW // 128)
    ]
    for c in copies:
      c.wait()

    for r in range(6):
      pltpu.sync_copy(comp_v.at[pl.ds(r * TPW, TPW)],
                      out_hbm.at[pl.ds(r * B + base, TPW)])

  return gather_kernel(ids_t_flat, table_flat)


def _hyp_lca_dist(ax, ay, bx, by):
  tiny = 1e-30
  xn2 = jnp.maximum(ax * ax + ay * ay, tiny)
  rx = ax / xn2
  ry = ay / xn2
  r2 = (rx * rx + ry * ry) - 1.0
  ux = bx - rx
  uy = by - ry
  t = r2 / jnp.maximum(ux * ux + uy * uy, tiny)
  yix = t * ux + rx
  yiy = t * uy + ry
  xta = ax * yix + ay * yiy
  c = 2.0 * xta / jnp.maximum(yix * yix + yiy * yiy, tiny)
  ox = c * yix - ax
  oy = c * yiy - ay
  u2x = ox - rx
  u2y = oy - ry
  t2 = r2 / jnp.maximum(u2x * u2x + u2y * u2y, tiny)
  orx = t2 * u2x + rx
  ory = t2 * u2y + ry
  on2 = orx * orx + ory * ory
  denom = 1.0 + jnp.sqrt(jnp.maximum(1.0 - on2, 0.0))
  pn = jnp.minimum(jnp.sqrt(on2) / denom, np.float32(1.0 - 1e-7))
  return jnp.log((1.0 + pn) / (1.0 - pn))


def _tc_math_kernel(e_ref, sim_ref, scale_ref, out_ref):
  s = jnp.clip(scale_ref[0, 0], 0.01, MAX_SCALE)
  eps = 1e-12

  comps = []
  for j in range(3):
    x = e_ref[2 * j]
    y = e_ref[2 * j + 1]
    n = jnp.sqrt(x * x + y * y)
    f = s / jnp.maximum(n, eps)
    comps.append((x * f, y * f))
  (x1, y1), (x2, y2), (x3, y3) = comps

  inv_t = np.float32(1.0 / TEMPERATURE)
  a12 = _hyp_lca_dist(x1, y1, x2, y2) * inv_t
  a13 = _hyp_lca_dist(x1, y1, x3, y3) * inv_t
  a23 = _hyp_lca_dist(x2, y2, x3, y3) * inv_t

  m = jnp.maximum(jnp.maximum(a12, a13), a23)
  w12 = jnp.exp(a12 - m)
  w13 = jnp.exp(a13 - m)
  w23 = jnp.exp(a23 - m)
  z = w12 + w13 + w23

  s1 = sim_ref[0]
  s2 = sim_ref[1]
  s3 = sim_ref[2]
  w_ord = (s1 * w12 + s2 * w13 + s3 * w23) / z
  total = (s1 + s2 + s3) - w_ord
  out_ref[0, 0] = jnp.sum(total) / np.float32(B)


def kernel(triple_ids, similarities, embeddings, scale):
  ids = triple_ids.astype(jnp.int32).T.reshape(3 * B)
  comps = _sc_gather(ids, embeddings.reshape(2 * embeddings.shape[0]))

  e = comps.reshape(6, 128, 128)
  sim = similarities.T.reshape(3, 128, 128)
  sc = scale.reshape(1, 1)

  out = pl.pallas_call(
      _tc_math_kernel,
      out_shape=jax.ShapeDtypeStruct((1, 1), jnp.float32),
      out_specs=pl.BlockSpec(memory_space=pltpu.MemorySpace.SMEM),
  )(e, sim, sc)
  return out[0, 0]


# speedup vs baseline: 18.3012x; 18.3012x over previous
import functools

import jax
import jax.numpy as jnp
import numpy as np
from jax import lax
from jax.experimental import pallas as pl
from jax.experimental.pallas import tpu as pltpu
from jax.experimental.pallas import tpu_sc as plsc

B = 16384
TEMPERATURE = 0.05
MAX_SCALE = 1.0 - 0.001

NC = 2
NS = 16
NW = NC * NS
TPW = B // NW
IPW = 3 * TPW
IDX_ROWS = IPW // 128
CHUNKS = TPW // 16


def _sc_gather(ids_t_flat, ex, ey):
  mesh = plsc.VectorSubcoreMesh(
      core_axis_name="c", subcore_axis_name="s", num_cores=NC, num_subcores=NS)

  @functools.partial(
      pl.kernel,
      out_type=jax.ShapeDtypeStruct((6 * B,), jnp.float32),
      mesh=mesh,
      scratch_types=[
          pltpu.VMEM((IPW,), jnp.int32),
          pltpu.VMEM((6 * TPW,), jnp.float32),
          pltpu.SemaphoreType.DMA,
      ],
  )
  def gather_kernel(ids_hbm, ex_hbm, ey_hbm, out_hbm, ids_v, comp_v, sem):
    wid = lax.axis_index("s") * NC + lax.axis_index("c")
    base = wid * TPW
    for j in range(3):
      pltpu.sync_copy(ids_hbm.at[pl.ds(j * B + base, TPW)],
                      ids_v.at[pl.ds(j * TPW, TPW)])

    copies = []
    for j in range(3):
      for q in range(TPW // 128):
        idx = ids_v.at[pl.ds(j * TPW + q * 128, 128)]
        copies.append(pltpu.async_copy(
            ex_hbm.at[idx],
            comp_v.at[pl.ds((2 * j) * TPW + q * 128, 128)], sem))
        copies.append(pltpu.async_copy(
            ey_hbm.at[idx],
            comp_v.at[pl.ds((2 * j + 1) * TPW + q * 128, 128)], sem))
    for c in copies:
      c.wait()

    for r in range(6):
      pltpu.sync_copy(comp_v.at[pl.ds(r * TPW, TPW)],
                      out_hbm.at[pl.ds(r * B + base, TPW)])

  return gather_kernel(ids_t_flat, ex, ey)


def _hyp_lca_dist(ax, ay, bx, by):
  tiny = 1e-30
  xn2 = jnp.maximum(ax * ax + ay * ay, tiny)
  rx = ax / xn2
  ry = ay / xn2
  r2 = (rx * rx + ry * ry) - 1.0
  ux = bx - rx
  uy = by - ry
  t = r2 / jnp.maximum(ux * ux + uy * uy, tiny)
  yix = t * ux + rx
  yiy = t * uy + ry
  xta = ax * yix + ay * yiy
  c = 2.0 * xta / jnp.maximum(yix * yix + yiy * yiy, tiny)
  ox = c * yix - ax
  oy = c * yiy - ay
  u2x = ox - rx
  u2y = oy - ry
  t2 = r2 / jnp.maximum(u2x * u2x + u2y * u2y, tiny)
  orx = t2 * u2x + rx
  ory = t2 * u2y + ry
  on2 = orx * orx + ory * ory
  denom = 1.0 + jnp.sqrt(jnp.maximum(1.0 - on2, 0.0))
  pn = jnp.minimum(jnp.sqrt(on2) / denom, np.float32(1.0 - 1e-7))
  return jnp.log((1.0 + pn) / (1.0 - pn))


def _tc_math_kernel(e_ref, sim_ref, scale_ref, out_ref):
  s = jnp.clip(scale_ref[0, 0], 0.01, MAX_SCALE)
  eps = 1e-12

  comps = []
  for j in range(3):
    x = e_ref[2 * j]
    y = e_ref[2 * j + 1]
    n = jnp.sqrt(x * x + y * y)
    f = s / jnp.maximum(n, eps)
    comps.append((x * f, y * f))
  (x1, y1), (x2, y2), (x3, y3) = comps

  inv_t = np.float32(1.0 / TEMPERATURE)
  a12 = _hyp_lca_dist(x1, y1, x2, y2) * inv_t
  a13 = _hyp_lca_dist(x1, y1, x3, y3) * inv_t
  a23 = _hyp_lca_dist(x2, y2, x3, y3) * inv_t

  m = jnp.maximum(jnp.maximum(a12, a13), a23)
  w12 = jnp.exp(a12 - m)
  w13 = jnp.exp(a13 - m)
  w23 = jnp.exp(a23 - m)
  z = w12 + w13 + w23

  s1 = sim_ref[0]
  s2 = sim_ref[1]
  s3 = sim_ref[2]
  w_ord = (s1 * w12 + s2 * w13 + s3 * w23) / z
  total = (s1 + s2 + s3) - w_ord
  out_ref[0, 0] = jnp.sum(total) / np.float32(B)


def kernel(triple_ids, similarities, embeddings, scale):
  ids = triple_ids.astype(jnp.int32).T.reshape(3 * B)
  comps = _sc_gather(ids, embeddings[:, 0], embeddings[:, 1])

  e = comps.reshape(6, 128, 128)
  sim = similarities.T.reshape(3, 128, 128)
  sc = scale.reshape(1, 1)

  out = pl.pallas_call(
      _tc_math_kernel,
      out_shape=jax.ShapeDtypeStruct((1, 1), jnp.float32),
      out_specs=pl.BlockSpec(memory_space=pltpu.MemorySpace.SMEM),
  )(e, sim, sc)
  return out[0, 0]


# speedup vs baseline: 34.5060x; 1.8854x over previous
import functools

import jax
import jax.numpy as jnp
import numpy as np
from jax import lax
from jax.experimental import pallas as pl
from jax.experimental.pallas import tpu as pltpu
from jax.experimental.pallas import tpu_sc as plsc

B = 16384
TEMPERATURE = 0.05
MAX_SCALE = 1.0 - 0.001

NC = 2
NS = 16
NW = NC * NS
TPW = B // NW
IPW = 3 * TPW
IDX_ROWS = IPW // 128
CHUNKS = TPW // 16


def _sc_gather(ids_t_flat, exy, n_nodes):
  mesh = plsc.VectorSubcoreMesh(
      core_axis_name="c", subcore_axis_name="s", num_cores=NC, num_subcores=NS)

  @functools.partial(
      pl.kernel,
      out_type=jax.ShapeDtypeStruct((6 * B,), jnp.float32),
      mesh=mesh,
      scratch_types=[
          pltpu.VMEM((2 * IPW,), jnp.int32),
          pltpu.VMEM((6 * TPW,), jnp.float32),
          pltpu.SemaphoreType.DMA,
      ],
  )
  def gather_kernel(ids_hbm, exy_hbm, out_hbm, ids_v, comp_v, sem):
    wid = lax.axis_index("s") * NC + lax.axis_index("c")
    base = wid * TPW
    for j in range(3):
      pltpu.sync_copy(ids_hbm.at[pl.ds(j * B + base, TPW)],
                      ids_v.at[pl.ds(j * TPW, TPW)])

    def body(p, carry):
      idv = ids_v[pl.ds(p * 16, 16)]
      ids_v[pl.ds(IPW + p * 16, 16)] = idv + n_nodes
      return carry

    lax.fori_loop(0, IPW // 16, body, 0, unroll=False)

    copies = []
    for j in range(3):
      for q in range(TPW // 128):
        off = j * TPW + q * 128
        copies.append(pltpu.async_copy(
            exy_hbm.at[ids_v.at[pl.ds(off, 128)]],
            comp_v.at[pl.ds((2 * j) * TPW + q * 128, 128)], sem))
        copies.append(pltpu.async_copy(
            exy_hbm.at[ids_v.at[pl.ds(IPW + off, 128)]],
            comp_v.at[pl.ds((2 * j + 1) * TPW + q * 128, 128)], sem))
    for c in copies:
      c.wait()

    for r in range(6):
      pltpu.sync_copy(comp_v.at[pl.ds(r * TPW, TPW)],
                      out_hbm.at[pl.ds(r * B + base, TPW)])

  return gather_kernel(ids_t_flat, exy)


def _hyp_lca_dist(ax, ay, bx, by):
  tiny = 1e-30
  xn2 = jnp.maximum(ax * ax + ay * ay, tiny)
  rx = ax / xn2
  ry = ay / xn2
  r2 = (rx * rx + ry * ry) - 1.0
  ux = bx - rx
  uy = by - ry
  t = r2 / jnp.maximum(ux * ux + uy * uy, tiny)
  yix = t * ux + rx
  yiy = t * uy + ry
  xta = ax * yix + ay * yiy
  c = 2.0 * xta / jnp.maximum(yix * yix + yiy * yiy, tiny)
  ox = c * yix - ax
  oy = c * yiy - ay
  u2x = ox - rx
  u2y = oy - ry
  t2 = r2 / jnp.maximum(u2x * u2x + u2y * u2y, tiny)
  orx = t2 * u2x + rx
  ory = t2 * u2y + ry
  on2 = orx * orx + ory * ory
  denom = 1.0 + jnp.sqrt(jnp.maximum(1.0 - on2, 0.0))
  pn = jnp.minimum(jnp.sqrt(on2) / denom, np.float32(1.0 - 1e-7))
  return jnp.log((1.0 + pn) / (1.0 - pn))


def _tc_math_kernel(e_ref, sim_ref, scale_ref, out_ref):
  s = jnp.clip(scale_ref[0, 0], 0.01, MAX_SCALE)
  eps = 1e-12

  comps = []
  for j in range(3):
    x = e_ref[2 * j]
    y = e_ref[2 * j + 1]
    n = jnp.sqrt(x * x + y * y)
    f = s / jnp.maximum(n, eps)
    comps.append((x * f, y * f))
  (x1, y1), (x2, y2), (x3, y3) = comps

  inv_t = np.float32(1.0 / TEMPERATURE)
  a12 = _hyp_lca_dist(x1, y1, x2, y2) * inv_t
  a13 = _hyp_lca_dist(x1, y1, x3, y3) * inv_t
  a23 = _hyp_lca_dist(x2, y2, x3, y3) * inv_t

  m = jnp.maximum(jnp.maximum(a12, a13), a23)
  w12 = jnp.exp(a12 - m)
  w13 = jnp.exp(a13 - m)
  w23 = jnp.exp(a23 - m)
  z = w12 + w13 + w23

  s1 = sim_ref[0]
  s2 = sim_ref[1]
  s3 = sim_ref[2]
  w_ord = (s1 * w12 + s2 * w13 + s3 * w23) / z
  total = (s1 + s2 + s3) - w_ord
  out_ref[0, 0] = jnp.sum(total) / np.float32(B)


def kernel(triple_ids, similarities, embeddings, scale):
  n_nodes = embeddings.shape[0]
  ids = triple_ids.astype(jnp.int32).T.reshape(3 * B)
  comps = _sc_gather(ids, embeddings.T.reshape(2 * n_nodes), n_nodes)

  e = comps.reshape(6, 128, 128)
  sim = similarities.T.reshape(3, 128, 128)
  sc = scale.reshape(1, 1)

  out = pl.pallas_call(
      _tc_math_kernel,
      out_shape=jax.ShapeDtypeStruct((1, 1), jnp.float32),
      out_specs=pl.BlockSpec(memory_space=pltpu.MemorySpace.SMEM),
  )(e, sim, sc)
  return out[0, 0]


# speedup vs baseline: 35.6567x; 1.0333x over previous
import functools

import jax
import jax.numpy as jnp
import numpy as np
from jax import lax
from jax.experimental import pallas as pl
from jax.experimental.pallas import tpu as pltpu
from jax.experimental.pallas import tpu_sc as plsc

B = 16384
TEMPERATURE = 0.05
MAX_SCALE = 1.0 - 0.001

NC = 2
NS = 16
NW = NC * NS
TPW = B // NW
IPW = 3 * TPW
IDX_ROWS = IPW // 128
CHUNKS = TPW // 16


def _sc_gather(ids_t_flat, exy, n_nodes):
  mesh = plsc.VectorSubcoreMesh(
      core_axis_name="c", subcore_axis_name="s", num_cores=NC, num_subcores=NS)

  @functools.partial(
      pl.kernel,
      out_type=jax.ShapeDtypeStruct((6 * B,), jnp.float32),
      mesh=mesh,
      scratch_types=[
          pltpu.VMEM((2 * IPW,), jnp.int32),
          pltpu.VMEM((6 * TPW,), jnp.float32),
          pltpu.SemaphoreType.DMA,
      ],
  )
  def gather_kernel(ids_hbm, exy_hbm, out_hbm, ids_v, comp_v, sem):
    wid = lax.axis_index("s") * NC + lax.axis_index("c")
    base = wid * TPW
    stages = [
        pltpu.async_copy(ids_hbm.at[pl.ds(j * B + base, TPW)],
                         ids_v.at[pl.ds(j * TPW, TPW)], sem)
        for j in range(3)
    ]
    for st in stages:
      st.wait()

    def body(p, carry):
      idv = ids_v[pl.ds(p * 16, 16)]
      ids_v[pl.ds(IPW + p * 16, 16)] = idv + n_nodes
      return carry

    lax.fori_loop(0, IPW // 16, body, 0, unroll=False)

    copies = []
    for j in range(3):
      for q in range(TPW // 128):
        off = j * TPW + q * 128
        copies.append(pltpu.async_copy(
            exy_hbm.at[ids_v.at[pl.ds(off, 128)]],
            comp_v.at[pl.ds((2 * j) * TPW + q * 128, 128)], sem))
        copies.append(pltpu.async_copy(
            exy_hbm.at[ids_v.at[pl.ds(IPW + off, 128)]],
            comp_v.at[pl.ds((2 * j + 1) * TPW + q * 128, 128)], sem))
    for c in copies:
      c.wait()

    outs = [
        pltpu.async_copy(comp_v.at[pl.ds(r * TPW, TPW)],
                         out_hbm.at[pl.ds(r * B + base, TPW)], sem)
        for r in range(6)
    ]
    for o in outs:
      o.wait()

  return gather_kernel(ids_t_flat, exy)


def _hyp_lca_dist(ax, ay, bx, by):
  tiny = 1e-30
  xn2 = jnp.maximum(ax * ax + ay * ay, tiny)
  rx = ax / xn2
  ry = ay / xn2
  r2 = (rx * rx + ry * ry) - 1.0
  ux = bx - rx
  uy = by - ry
  t = r2 / jnp.maximum(ux * ux + uy * uy, tiny)
  yix = t * ux + rx
  yiy = t * uy + ry
  xta = ax * yix + ay * yiy
  c = 2.0 * xta / jnp.maximum(yix * yix + yiy * yiy, tiny)
  ox = c * yix - ax
  oy = c * yiy - ay
  u2x = ox - rx
  u2y = oy - ry
  t2 = r2 / jnp.maximum(u2x * u2x + u2y * u2y, tiny)
  orx = t2 * u2x + rx
  ory = t2 * u2y + ry
  on2 = orx * orx + ory * ory
  denom = 1.0 + jnp.sqrt(jnp.maximum(1.0 - on2, 0.0))
  pn = jnp.minimum(jnp.sqrt(on2) / denom, np.float32(1.0 - 1e-7))
  return jnp.log((1.0 + pn) / (1.0 - pn))


def _tc_math_kernel(e_ref, sim_ref, scale_ref, out_ref):
  s = jnp.clip(scale_ref[0, 0], 0.01, MAX_SCALE)
  eps = 1e-12

  comps = []
  for j in range(3):
    x = e_ref[2 * j]
    y = e_ref[2 * j + 1]
    n = jnp.sqrt(x * x + y * y)
    f = s / jnp.maximum(n, eps)
    comps.append((x * f, y * f))
  (x1, y1), (x2, y2), (x3, y3) = comps

  inv_t = np.float32(1.0 / TEMPERATURE)
  a12 = _hyp_lca_dist(x1, y1, x2, y2) * inv_t
  a13 = _hyp_lca_dist(x1, y1, x3, y3) * inv_t
  a23 = _hyp_lca_dist(x2, y2, x3, y3) * inv_t

  m = jnp.maximum(jnp.maximum(a12, a13), a23)
  w12 = jnp.exp(a12 - m)
  w13 = jnp.exp(a13 - m)
  w23 = jnp.exp(a23 - m)
  z = w12 + w13 + w23

  s1 = sim_ref[0]
  s2 = sim_ref[1]
  s3 = sim_ref[2]
  w_ord = (s1 * w12 + s2 * w13 + s3 * w23) / z
  total = (s1 + s2 + s3) - w_ord
  out_ref[0, 0] = jnp.sum(total) / np.float32(B)


def kernel(triple_ids, similarities, embeddings, scale):
  n_nodes = embeddings.shape[0]
  ids = triple_ids.astype(jnp.int32).T.reshape(3 * B)
  comps = _sc_gather(ids, embeddings.T.reshape(2 * n_nodes), n_nodes)

  e = comps.reshape(6, 128, 128)
  sim = similarities.T.reshape(3, 128, 128)
  sc = scale.reshape(1, 1)

  out = pl.pallas_call(
      _tc_math_kernel,
      out_shape=jax.ShapeDtypeStruct((1, 1), jnp.float32),
      out_specs=pl.BlockSpec(memory_space=pltpu.MemorySpace.SMEM),
  )(e, sim, sc)
  return out[0, 0]


# speedup vs baseline: 35.7881x; 1.0037x over previous
import functools

import jax
import jax.numpy as jnp
import numpy as np
from jax import lax
from jax.experimental import pallas as pl
from jax.experimental.pallas import tpu as pltpu
from jax.experimental.pallas import tpu_sc as plsc

B = 16384
TEMPERATURE = 0.05
MAX_SCALE = 1.0 - 0.001

NC = 2
NS = 16
NW = NC * NS
TPW = B // NW
IPW = 3 * TPW
IDX_ROWS = IPW // 128
CHUNKS = TPW // 16


def _sc_gather(ids_t_flat, exy, n_nodes):
  mesh = plsc.VectorSubcoreMesh(
      core_axis_name="c", subcore_axis_name="s", num_cores=NC, num_subcores=NS)

  @functools.partial(
      pl.kernel,
      out_type=jax.ShapeDtypeStruct((6 * B,), jnp.float32),
      mesh=mesh,
      scratch_types=[
          pltpu.VMEM((IPW,), jnp.int32),
          pltpu.VMEM((6 * TPW,), jnp.float32),
          pltpu.SemaphoreType.DMA,
      ],
  )
  def gather_kernel(ids_hbm, exy_hbm, out_hbm, ids_v, comp_v, sem):
    wid = lax.axis_index("s") * NC + lax.axis_index("c")
    base = wid * TPW
    stages = [
        pltpu.async_copy(ids_hbm.at[pl.ds(j * B + base, TPW)],
                         ids_v.at[pl.ds(j * TPW, TPW)], sem)
        for j in range(3)
    ]
    for st in stages:
      st.wait()

    ey_hbm = exy_hbm.at[pl.ds(n_nodes, n_nodes)]
    copies = []
    for j in range(3):
      for q in range(TPW // 128):
        off = j * TPW + q * 128
        idx = ids_v.at[pl.ds(off, 128)]
        copies.append(pltpu.async_copy(
            exy_hbm.at[idx],
            comp_v.at[pl.ds((2 * j) * TPW + q * 128, 128)], sem))
        copies.append(pltpu.async_copy(
            ey_hbm.at[idx],
            comp_v.at[pl.ds((2 * j + 1) * TPW + q * 128, 128)], sem))
    for c in copies:
      c.wait()

    outs = [
        pltpu.async_copy(comp_v.at[pl.ds(r * TPW, TPW)],
                         out_hbm.at[pl.ds(r * B + base, TPW)], sem)
        for r in range(6)
    ]
    for o in outs:
      o.wait()

  return gather_kernel(ids_t_flat, exy)


def _hyp_lca_dist(ax, ay, bx, by):
  tiny = 1e-30
  xn2 = jnp.maximum(ax * ax + ay * ay, tiny)
  rx = ax / xn2
  ry = ay / xn2
  r2 = (rx * rx + ry * ry) - 1.0
  ux = bx - rx
  uy = by - ry
  t = r2 / jnp.maximum(ux * ux + uy * uy, tiny)
  yix = t * ux + rx
  yiy = t * uy + ry
  xta = ax * yix + ay * yiy
  c = 2.0 * xta / jnp.maximum(yix * yix + yiy * yiy, tiny)
  ox = c * yix - ax
  oy = c * yiy - ay
  u2x = ox - rx
  u2y = oy - ry
  t2 = r2 / jnp.maximum(u2x * u2x + u2y * u2y, tiny)
  orx = t2 * u2x + rx
  ory = t2 * u2y + ry
  on2 = orx * orx + ory * ory
  denom = 1.0 + jnp.sqrt(jnp.maximum(1.0 - on2, 0.0))
  pn = jnp.minimum(jnp.sqrt(on2) / denom, np.float32(1.0 - 1e-7))
  return jnp.log((1.0 + pn) / (1.0 - pn))


def _tc_math_kernel(e_ref, sim_ref, scale_ref, out_ref):
  s = jnp.clip(scale_ref[0, 0], 0.01, MAX_SCALE)
  eps = 1e-12

  comps = []
  for j in range(3):
    x = e_ref[2 * j]
    y = e_ref[2 * j + 1]
    n = jnp.sqrt(x * x + y * y)
    f = s / jnp.maximum(n, eps)
    comps.append((x * f, y * f))
  (x1, y1), (x2, y2), (x3, y3) = comps

  inv_t = np.float32(1.0 / TEMPERATURE)
  a12 = _hyp_lca_dist(x1, y1, x2, y2) * inv_t
  a13 = _hyp_lca_dist(x1, y1, x3, y3) * inv_t
  a23 = _hyp_lca_dist(x2, y2, x3, y3) * inv_t

  m = jnp.maximum(jnp.maximum(a12, a13), a23)
  w12 = jnp.exp(a12 - m)
  w13 = jnp.exp(a13 - m)
  w23 = jnp.exp(a23 - m)
  z = w12 + w13 + w23

  s1 = sim_ref[0]
  s2 = sim_ref[1]
  s3 = sim_ref[2]
  w_ord = (s1 * w12 + s2 * w13 + s3 * w23) / z
  total = (s1 + s2 + s3) - w_ord
  out_ref[0, 0] = jnp.sum(total) / np.float32(B)


def kernel(triple_ids, similarities, embeddings, scale):
  n_nodes = embeddings.shape[0]
  ids = triple_ids.astype(jnp.int32).T.reshape(3 * B)
  comps = _sc_gather(ids, embeddings.T.reshape(2 * n_nodes), n_nodes)

  e = comps.reshape(6, 128, 128)
  sim = similarities.T.reshape(3, 128, 128)
  sc = scale.reshape(1, 1)

  out = pl.pallas_call(
      _tc_math_kernel,
      out_shape=jax.ShapeDtypeStruct((1, 1), jnp.float32),
      out_specs=pl.BlockSpec(memory_space=pltpu.MemorySpace.SMEM),
  )(e, sim, sc)
  return out[0, 0]
